# baseline (speedup 1.0000x reference)
import functools
import math

import jax
import jax.numpy as jnp
from jax import lax
from jax.experimental import pallas as pl
from jax.experimental.pallas import tpu as pltpu
from jax.experimental.pallas import tpu_sc as plsc

D_MODEL = 64
PADDED = 128
SCALE = math.sqrt(D_MODEL)
NUM_CORES = 2
NUM_SUBCORES = 16
NUM_WORKERS = NUM_CORES * NUM_SUBCORES
LANES = 16
NI = 4
NO = 2
CHUNK = 128


@functools.lru_cache(maxsize=None)
def _build(n_tok: int):
    per_w = n_tok // NUM_WORKERS
    n_chunks = per_w // CHUNK
    assert per_w * NUM_WORKERS == n_tok and n_chunks * CHUNK == per_w
    assert (n_chunks - 2 * NI) % NI == 0 and n_chunks >= 3 * NI

    mesh = plsc.VectorSubcoreMesh(
        core_axis_name="c", subcore_axis_name="s",
        num_cores=NUM_CORES, num_subcores=NUM_SUBCORES,
    )

    @functools.partial(
        pl.kernel,
        out_type=jax.ShapeDtypeStruct((n_tok, D_MODEL), jnp.float32),
        mesh=mesh,
        compiler_params=pltpu.CompilerParams(use_tc_tiling_on_sc=True),
        scratch_types=[
            pltpu.VMEM((per_w,), jnp.int32),
            pltpu.VMEM((NI, CHUNK, PADDED), jnp.float32),
            pltpu.VMEM((NO, CHUNK, D_MODEL), jnp.float32),
            pltpu.SemaphoreType.DMA((NI,)),
            pltpu.SemaphoreType.DMA((NO,)),
        ],
    )
    def emb(idx_hbm, table_hbm, out_hbm, idx_v, ibuf, obuf, gsem, osem):
        wid = lax.axis_index("s") * NUM_CORES + lax.axis_index("c")
        base = wid * per_w

        pltpu.sync_copy(idx_hbm.at[pl.ds(base, per_w)], idx_v)

        def start_gather(i, b):
            pltpu.async_copy(table_hbm.at[idx_v.at[pl.ds(i * CHUNK, CHUNK)]],
                             ibuf.at[b], gsem.at[b])

        def wait_gather(i, b):
            pltpu.make_async_copy(table_hbm.at[idx_v.at[pl.ds(i * CHUNK, CHUNK)]],
                                  ibuf.at[b], gsem.at[b]).wait()

        def scale(b, o):
            @pl.loop(0, CHUNK, unroll=8)
            def row(t):
                for j in range(D_MODEL // LANES):
                    sl = pl.ds(j * LANES, LANES)
                    obuf[o, t, sl] = ibuf[b, t, sl] * SCALE

        def start_write(i, o):
            pltpu.async_copy(obuf.at[o],
                             out_hbm.at[pl.ds(base + i * CHUNK, CHUNK)],
                             osem.at[o])

        def wait_write(i, o):
            pltpu.make_async_copy(obuf.at[o],
                                  out_hbm.at[pl.ds(base + i * CHUNK, CHUNK)],
                                  osem.at[o]).wait()

        for i in range(NI):
            start_gather(i, i)

        for i in range(NI):
            wait_gather(i, i)
            if i >= NO:
                wait_write(i - NO, i % NO)
            scale(i, i % NO)
            start_write(i, i % NO)
            start_gather(i + NI, i)

        n_groups = (n_chunks - 2 * NI) // NI

        @pl.loop(0, n_groups)
        def group(g):
            for k in range(NI):
                i = NI + g * NI + k
                wait_gather(i, k)
                wait_write(i - NO, k % NO)
                scale(k, k % NO)
                start_write(i, k % NO)
                start_gather(i + NI, k)

        for k in range(NI):
            i = n_chunks - NI + k
            wait_gather(i, k)
            wait_write(i - NO, k % NO)
            scale(k, k % NO)
            start_write(i, k % NO)

        for k in range(NO):
            i = n_chunks - NO + k
            wait_write(i, i % NO)

    return emb


def kernel(token_id_tensor, embedding_table):
    b, s = token_id_tensor.shape
    idx = token_id_tensor.astype(jnp.int32).reshape(b * s)
    tab = jnp.pad(embedding_table, ((0, 0), (0, PADDED - D_MODEL)))
    out = _build(b * s)(idx, tab)
    return out.reshape(b, s, D_MODEL)


# speedup vs baseline: 1.0124x; 1.0124x over previous
import functools
import math

import jax
import jax.numpy as jnp
from jax import lax
from jax.experimental import pallas as pl
from jax.experimental.pallas import tpu as pltpu
from jax.experimental.pallas import tpu_sc as plsc

D_MODEL = 64
PADDED = 128
SCALE = math.sqrt(D_MODEL)
NUM_CORES = 2
NUM_SUBCORES = 16
NUM_WORKERS = NUM_CORES * NUM_SUBCORES
LANES = 16
NBUF = 2
CHUNK = 128


@functools.lru_cache(maxsize=None)
def _build(n_tok: int):
    per_w = n_tok // NUM_WORKERS
    n_chunks = per_w // CHUNK
    assert per_w * NUM_WORKERS == n_tok and n_chunks * CHUNK == per_w

    mesh = plsc.VectorSubcoreMesh(
        core_axis_name="c", subcore_axis_name="s",
        num_cores=NUM_CORES, num_subcores=NUM_SUBCORES,
    )

    @functools.partial(
        pl.kernel,
        out_type=jax.ShapeDtypeStruct((n_tok, D_MODEL), jnp.float32),
        mesh=mesh,
        compiler_params=pltpu.CompilerParams(use_tc_tiling_on_sc=True),
        scratch_types=[
            pltpu.VMEM((per_w,), jnp.int32),
            pltpu.VMEM((NBUF, CHUNK, PADDED), jnp.float32),
            pltpu.VMEM((NBUF, CHUNK, D_MODEL), jnp.float32),
            pltpu.SemaphoreType.DMA((NBUF,)),
            pltpu.SemaphoreType.DMA((NBUF,)),
        ],
    )
    def emb(idx_hbm, table_hbm, out_hbm, idx_v, ibuf, obuf, gsem, osem):
        wid = lax.axis_index("s") * NUM_CORES + lax.axis_index("c")
        base = wid * per_w

        pltpu.sync_copy(idx_hbm.at[pl.ds(base, per_w)], idx_v)

        def start_gather(i, b):
            pltpu.async_copy(table_hbm.at[idx_v.at[pl.ds(i * CHUNK, CHUNK)]],
                             ibuf.at[b], gsem.at[b])

        def wait_gather(i, b):
            pltpu.make_async_copy(table_hbm.at[idx_v.at[pl.ds(i * CHUNK, CHUNK)]],
                                  ibuf.at[b], gsem.at[b]).wait()

        def scale(b):
            def row(t, c):
                for j in range(D_MODEL // LANES):
                    sl = pl.ds(j * LANES, LANES)
                    obuf[b, t, sl] = ibuf[b, t, sl] * SCALE
                return c
            lax.fori_loop(0, CHUNK, row, 0)

        def start_write(i, b):
            pltpu.async_copy(obuf.at[b],
                             out_hbm.at[pl.ds(base + i * CHUNK, CHUNK)],
                             osem.at[b])

        def wait_write(i, b):
            pltpu.make_async_copy(obuf.at[b],
                                  out_hbm.at[pl.ds(base + i * CHUNK, CHUNK)],
                                  osem.at[b]).wait()

        for b in range(NBUF):
            start_gather(b, b)

        for b in range(NBUF):
            wait_gather(b, b)
            scale(b)
            start_gather(b + NBUF, b)
            start_write(b, b)

        n_groups = n_chunks // NBUF
        assert n_groups * NBUF == n_chunks and n_groups >= 3

        @pl.loop(1, n_groups - 1)
        def group(g):
            for b in range(NBUF):
                i = g * NBUF + b
                wait_gather(i, b)
                wait_write(i - NBUF, b)
                scale(b)
                start_gather(i + NBUF, b)
                start_write(i, b)

        for b in range(NBUF):
            i = (n_groups - 1) * NBUF + b
            wait_gather(i, b)
            wait_write(i - NBUF, b)
            scale(b)
            start_write(i, b)

        for b in range(NBUF):
            i = (n_groups - 1) * NBUF + b
            wait_write(i, b)

    return emb


def kernel(token_id_tensor, embedding_table):
    b, s = token_id_tensor.shape
    idx = token_id_tensor.astype(jnp.int32).reshape(b * s)
    tab = jnp.pad(embedding_table, ((0, 0), (0, PADDED - D_MODEL)))
    out = _build(b * s)(idx, tab)
    return out.reshape(b, s, D_MODEL)
